# Initial kernel scaffold; baseline (speedup 1.0000x reference)
import jax
import jax.numpy as jnp
from jax.experimental import pallas as pl


def kernel(x, edge_index, aw0, aw1, W1, b1, W2, b2, Wl, bl):
    raise NotImplementedError("write your pallas kernel here")


# speedup vs baseline: 2.2938x; 2.2938x over previous
import jax
import jax.numpy as jnp
from jax import lax
from jax.experimental import pallas as pl
from jax.experimental.pallas import tpu as pltpu
from jax.experimental.pallas import tpu_sc as plsc

_N = 10000
_NPAD = 10240
_E = 160000
_FC = 128
_NPLANE = 4
_EB = 80
_NBATCH = _E // (16 * _EB)
_RPT = _NPAD // 16
_WB = 128

_BM = 1000


def _seg_body(p0, p1, p2, p3, src_hbm, dst_hbm, ew_hbm,
              o0, o1, o2, o3,
              idx_s, idx_d, ewb, rows, zbuf, stage, acc, sem):
    c = lax.axis_index("c")
    s = lax.axis_index("s")
    planes = (p0, p1, p2, p3)
    outs = (o0, o1, o2, o3)

    zero16 = jnp.zeros((16,), jnp.float32)

    def zrow(i, carry):
        for j in range(8):
            zbuf[i, pl.ds(j * 16, 16)] = zero16
        return carry

    lax.fori_loop(0, _WB, zrow, 0)

    row0 = s * _RPT
    ebase = s * (_E // 16)

    for k in range(_NPLANE):
        @pl.when((k // 2) == c)
        def _process():
            for m in range(_RPT // _WB):
                pltpu.sync_copy(zbuf, acc.at[pl.ds(row0 + m * _WB, _WB)])
            plsc.subcore_barrier()

            def batch(j, carry):
                off = ebase + j * _EB
                pltpu.sync_copy(src_hbm.at[pl.ds(off, _EB)], idx_s)
                pltpu.sync_copy(dst_hbm.at[pl.ds(off, _EB)], idx_d)
                pltpu.sync_copy(ew_hbm.at[pl.ds(off, _EB)], ewb)
                pltpu.async_copy(planes[k].at[idx_s], rows, sem).wait()

                def mulgrp(g, cc):
                    wvec = ewb[pl.ds(g * 16, 16)]
                    for i16 in range(16):
                        wv = jnp.broadcast_to(wvec[i16:i16 + 1], (16,))
                        r = g * 16 + i16
                        for jj in range(8):
                            rows[r, pl.ds(jj * 16, 16)] = (
                                rows[r, pl.ds(jj * 16, 16)] * wv)
                    return cc

                lax.fori_loop(0, _EB // 16, mulgrp, 0)
                pltpu.sync_copy(rows, acc.at[idx_d], add=True)
                return carry

            lax.fori_loop(0, _NBATCH, batch, 0)
            plsc.subcore_barrier()

            for m in range(_RPT // _WB):
                r0 = row0 + m * _WB
                pltpu.sync_copy(acc.at[pl.ds(r0, _WB)], stage)
                pltpu.sync_copy(stage, outs[k].at[pl.ds(r0, _WB)])


_seg = pl.kernel(
    _seg_body,
    out_type=[jax.ShapeDtypeStruct((_NPAD, _FC), jnp.float32)] * _NPLANE,
    mesh=plsc.VectorSubcoreMesh(core_axis_name="c", subcore_axis_name="s"),
    scratch_types=[
        pltpu.VMEM((_EB,), jnp.int32),
        pltpu.VMEM((_EB,), jnp.int32),
        pltpu.VMEM((_EB,), jnp.float32),
        pltpu.VMEM((_EB, _FC), jnp.float32),
        pltpu.VMEM((_WB, _FC), jnp.float32),
        pltpu.VMEM((_WB, _FC), jnp.float32),
        pltpu.VMEM_SHARED((_NPAD, _FC), jnp.float32),
        pltpu.SemaphoreType.DMA,
    ],
)


def _leaky(x):
    return jnp.where(x >= 0, x, 0.01 * x)


def _mm_in_body(x_ref, w_ref, o0, o1, o2, o3):
    y = jnp.dot(x_ref[...], w_ref[...], preferred_element_type=jnp.float32)
    o0[...] = y[:, 0:128]
    o1[...] = y[:, 128:256]
    o2[...] = y[:, 256:384]
    o3[...] = y[:, 384:512]


def _mm_mid_body(a0, a1, a2, a3, b_ref, w_ref, o0, o1, o2, o3):
    h = jnp.concatenate([a0[...], a1[...], a2[...], a3[...]], axis=1)
    h = _leaky(h + b_ref[...])
    y = jnp.dot(h, w_ref[...], preferred_element_type=jnp.float32)
    o0[...] = y[:, 0:128]
    o1[...] = y[:, 128:256]
    o2[...] = y[:, 256:384]
    o3[...] = y[:, 384:512]


def _mm_out_body(a0, a1, a2, a3, b_ref, w_ref, bl_ref, o_ref):
    h = jnp.concatenate([a0[...], a1[...], a2[...], a3[...]], axis=1)
    h = _leaky(h + b_ref[...])
    o_ref[...] = (jnp.dot(h, w_ref[...], preferred_element_type=jnp.float32)
                  + bl_ref[...])


def _plane_specs(n):
    return [pl.BlockSpec((_BM, _FC), lambda i: (i, 0)) for _ in range(n)]


def _mm_in(x, w):
    kin, h = x.shape[1], w.shape[1]
    return pl.pallas_call(
        _mm_in_body,
        grid=(_N // _BM,),
        in_specs=[
            pl.BlockSpec((_BM, kin), lambda i: (i, 0)),
            pl.BlockSpec((kin, h), lambda i: (0, 0)),
        ],
        out_specs=_plane_specs(_NPLANE),
        out_shape=[jax.ShapeDtypeStruct((_N, _FC), jnp.float32)] * _NPLANE,
    )(x, w)


def _mm_mid(a, b, w):
    h = w.shape[1]
    return pl.pallas_call(
        _mm_mid_body,
        grid=(_N // _BM,),
        in_specs=_plane_specs(_NPLANE) + [
            pl.BlockSpec((1, b.shape[1]), lambda i: (0, 0)),
            pl.BlockSpec((w.shape[0], h), lambda i: (0, 0)),
        ],
        out_specs=_plane_specs(_NPLANE),
        out_shape=[jax.ShapeDtypeStruct((_N, _FC), jnp.float32)] * _NPLANE,
    )(*a, b, w)


def _mm_out(a, b, w, bl):
    ncls = w.shape[1]
    return pl.pallas_call(
        _mm_out_body,
        grid=(_N // _BM,),
        in_specs=_plane_specs(_NPLANE) + [
            pl.BlockSpec((1, b.shape[1]), lambda i: (0, 0)),
            pl.BlockSpec((w.shape[0], ncls), lambda i: (0, 0)),
            pl.BlockSpec((1, ncls), lambda i: (0, 0)),
        ],
        out_specs=pl.BlockSpec((_BM, ncls), lambda i: (i, 0)),
        out_shape=jax.ShapeDtypeStruct((_N, ncls), jnp.float32),
    )(*a, b, w, bl)


def kernel(x, edge_index, aw0, aw1, W1, b1, W2, b2, Wl, bl):
    src = edge_index[0]
    dst = edge_index[1]
    p = _mm_in(x, W1)
    a = _seg(p[0], p[1], p[2], p[3], src, dst, aw0)
    q = _mm_mid(a, b1.reshape(1, -1), W2)
    g = _seg(q[0], q[1], q[2], q[3], src, dst, aw1)
    return _mm_out(g, b2.reshape(1, -1), Wl, bl.reshape(1, -1))


# speedup vs baseline: 2.7317x; 1.1909x over previous
import jax
import jax.numpy as jnp
from jax import lax
from jax.experimental import pallas as pl
from jax.experimental.pallas import tpu as pltpu
from jax.experimental.pallas import tpu_sc as plsc

_N = 10000
_NPAD = 10240
_E = 160000
_FC = 128
_NPLANE = 4
_EB = 80
_EROWS = 2048
_NB = _EROWS // 16
_NSEG = 8
_SB = _NB // _NSEG
_RPT = _NPAD // 16

_BM = 1000


def _seg_body(ps, src2, dst2, ew2, outs,
              srck, dstt, ewt, r0b, r1b, r2b, r3b,
              sg0, sg1, sg2, sg3, ss0, ss1, ss2, ss3, acc):
    c = lax.axis_index("c")
    s = lax.axis_index("s")
    rows = (r0b, r1b, r2b, r3b)
    sg = (sg0, sg1, sg2, sg3)
    ss = (ss0, ss1, ss2, ss3)

    rb = s * _NB
    row0 = s * _RPT
    zero16 = jnp.zeros((16,), jnp.float32)

    def chunk_body(kk, cc):
        k = c * 2 + kk
        koff = (k * _NPAD).astype(jnp.int32)

        def zr(i, c2):
            for j8 in range(8):
                r0b[i, pl.ds(j8 * 16, 16)] = zero16
            return c2

        lax.fori_loop(0, _EB, zr, 0)
        for m in range(_RPT // _EB):
            pltpu.sync_copy(r0b, acc.at[pl.ds(row0 + m * _EB, _EB)])
        plsc.subcore_barrier()

        def g_start(j, b):
            pltpu.async_copy(ps.at[srck.at[j]], rows[b], sg[b])

        def g_wait(b):
            pltpu.make_async_copy(ps.at[pl.ds(0, _EB)], rows[b],
                                  sg[b]).wait()

        def s_start(j, b):
            pltpu.async_copy(rows[b], acc.at[dstt.at[j]], ss[b], add=True)

        def s_wait(b):
            pltpu.make_async_copy(ps.at[pl.ds(0, _EB)], rows[b],
                                  ss[b]).wait()

        for seg in range(_NSEG):
            sb = rb + seg * _SB
            pltpu.sync_copy(src2.at[pl.ds(sb, _SB)], srck)
            pltpu.sync_copy(dst2.at[pl.ds(sb, _SB)], dstt)
            pltpu.sync_copy(ew2.at[pl.ds(sb, _SB)], ewt)

            kv = jnp.full((16,), 0, jnp.int32) + koff

            def addk(i, c2):
                for g in range(_EB // 16):
                    srck[i, pl.ds(g * 16, 16)] = (
                        srck[i, pl.ds(g * 16, 16)] + kv)
                return c2

            lax.fori_loop(0, _SB, addk, 0)

            g_start(0, 0)
            g_start(1, 1)
            g_start(2, 2)

            def ring_iter(it, c2):
                for b in range(4):
                    j = it * 4 + b
                    g_wait(b)

                    def grp(g, c3):
                        base = pl.multiple_of(g * 16, 16)
                        wvec = ewt[j, pl.ds(base, 16)]
                        buf = rows[b]
                        for i16 in range(16):
                            wv = jnp.broadcast_to(wvec[i16:i16 + 1], (16,))
                            r = base + i16
                            for j8 in range(8):
                                buf[r, pl.ds(j8 * 16, 16)] = (
                                    buf[r, pl.ds(j8 * 16, 16)] * wv)
                        return c3

                    lax.fori_loop(0, _EB // 16, grp, 0)

                    nb = (b + 3) % 4

                    @pl.when((j >= 1) & (j + 3 < _SB))
                    def _sw():
                        s_wait(nb)

                    s_start(j, b)

                    @pl.when(j + 3 < _SB)
                    def _gs():
                        g_start(j + 3, nb)
                return c2

            lax.fori_loop(0, _SB // 4, ring_iter, 0)
            for b in range(4):
                s_wait(b)

        plsc.subcore_barrier()
        for m in range(_RPT // _EB):
            r0 = row0 + m * _EB
            pltpu.sync_copy(acc.at[pl.ds(r0, _EB)], r0b)
            pltpu.sync_copy(r0b, outs.at[pl.ds(koff + r0, _EB)])
        return cc

    lax.fori_loop(0, _NPLANE // 2, chunk_body, 0)


_seg = pl.kernel(
    _seg_body,
    out_type=jax.ShapeDtypeStruct((_NPLANE * _NPAD, _FC), jnp.float32),
    mesh=plsc.VectorSubcoreMesh(core_axis_name="c", subcore_axis_name="s"),
    scratch_types=[
        pltpu.VMEM((_SB, _EB), jnp.int32),
        pltpu.VMEM((_SB, _EB), jnp.int32),
        pltpu.VMEM((_SB, _EB), jnp.float32),
        pltpu.VMEM((_EB, _FC), jnp.float32),
        pltpu.VMEM((_EB, _FC), jnp.float32),
        pltpu.VMEM((_EB, _FC), jnp.float32),
        pltpu.VMEM((_EB, _FC), jnp.float32),
        pltpu.SemaphoreType.DMA,
        pltpu.SemaphoreType.DMA,
        pltpu.SemaphoreType.DMA,
        pltpu.SemaphoreType.DMA,
        pltpu.SemaphoreType.DMA,
        pltpu.SemaphoreType.DMA,
        pltpu.SemaphoreType.DMA,
        pltpu.SemaphoreType.DMA,
        pltpu.VMEM_SHARED((_NPAD, _FC), jnp.float32),
    ],
)


def _leaky(x):
    return jnp.where(x >= 0, x, 0.01 * x)


def _mm_in_body(x_ref, w_ref, o_ref):
    y = jnp.dot(x_ref[...], w_ref[...], preferred_element_type=jnp.float32)
    for kk in range(_NPLANE):
        o_ref[kk] = y[:, kk * _FC:(kk + 1) * _FC]


def _mm_mid_body(a_ref, b_ref, w_ref, o_ref):
    h = jnp.concatenate([a_ref[kk] for kk in range(_NPLANE)], axis=1)
    h = _leaky(h + b_ref[...])
    y = jnp.dot(h, w_ref[...], preferred_element_type=jnp.float32)
    for kk in range(_NPLANE):
        o_ref[kk] = y[:, kk * _FC:(kk + 1) * _FC]


def _mm_out_body(a_ref, b_ref, w_ref, bl_ref, o_ref):
    h = jnp.concatenate([a_ref[kk] for kk in range(_NPLANE)], axis=1)
    h = _leaky(h + b_ref[...])
    o_ref[...] = (jnp.dot(h, w_ref[...], preferred_element_type=jnp.float32)
                  + bl_ref[...])


_STACK_SPEC = pl.BlockSpec((_NPLANE, _BM, _FC), lambda i: (0, i, 0))


def _mm_in(x, w):
    kin = x.shape[1]
    return pl.pallas_call(
        _mm_in_body,
        grid=(_N // _BM,),
        in_specs=[
            pl.BlockSpec((_BM, kin), lambda i: (i, 0)),
            pl.BlockSpec((kin, w.shape[1]), lambda i: (0, 0)),
        ],
        out_specs=_STACK_SPEC,
        out_shape=jax.ShapeDtypeStruct((_NPLANE, _NPAD, _FC), jnp.float32),
    )(x, w)


def _mm_mid(a, b, w):
    return pl.pallas_call(
        _mm_mid_body,
        grid=(_N // _BM,),
        in_specs=[
            _STACK_SPEC,
            pl.BlockSpec((1, b.shape[1]), lambda i: (0, 0)),
            pl.BlockSpec((w.shape[0], w.shape[1]), lambda i: (0, 0)),
        ],
        out_specs=_STACK_SPEC,
        out_shape=jax.ShapeDtypeStruct((_NPLANE, _NPAD, _FC), jnp.float32),
    )(a, b, w)


def _mm_out(a, b, w, bl):
    ncls = w.shape[1]
    return pl.pallas_call(
        _mm_out_body,
        grid=(_N // _BM,),
        in_specs=[
            _STACK_SPEC,
            pl.BlockSpec((1, b.shape[1]), lambda i: (0, 0)),
            pl.BlockSpec((w.shape[0], ncls), lambda i: (0, 0)),
            pl.BlockSpec((1, ncls), lambda i: (0, 0)),
        ],
        out_specs=pl.BlockSpec((_BM, ncls), lambda i: (i, 0)),
        out_shape=jax.ShapeDtypeStruct((_N, ncls), jnp.float32),
    )(a, b, w, bl)


def kernel(x, edge_index, aw0, aw1, W1, b1, W2, b2, Wl, bl):
    npad = _EROWS * _EB - _E
    src2 = jnp.pad(edge_index[0], (0, npad)).reshape(_EROWS, _EB)
    dst2 = jnp.pad(edge_index[1], (0, npad)).reshape(_EROWS, _EB)
    ew0 = jnp.pad(aw0, (0, npad)).reshape(_EROWS, _EB)
    ew1 = jnp.pad(aw1, (0, npad)).reshape(_EROWS, _EB)
    flat = (_NPLANE * _NPAD, _FC)
    stck = (_NPLANE, _NPAD, _FC)
    p = _mm_in(x, W1)
    a = _seg(p.reshape(flat), src2, dst2, ew0)
    q = _mm_mid(a.reshape(stck), b1.reshape(1, -1), W2)
    g = _seg(q.reshape(flat), src2, dst2, ew1)
    return _mm_out(g.reshape(stck), b2.reshape(1, -1), Wl, bl.reshape(1, -1))


# speedup vs baseline: 2.8017x; 1.0256x over previous
import jax
import jax.numpy as jnp
from jax import lax
from jax.experimental import pallas as pl
from jax.experimental.pallas import tpu as pltpu
from jax.experimental.pallas import tpu_sc as plsc

_N = 10000
_NPAD = 10240
_E = 160000
_FC = 128
_NPLANE = 4
_EB = 80
_EROWS = 2048
_NB = _EROWS // 16
_NSEG = 8
_SB = _NB // _NSEG
_RPT = _NPAD // 16

_BM = 1000


def _seg_body(ps, src2, dst2, ew2, outs,
              srck, dstt, ewt, r0b, r1b, r2b, r3b,
              sg0, sg1, sg2, sg3, ss0, ss1, ss2, ss3, acc):
    c = lax.axis_index("c")
    s = lax.axis_index("s")
    rows = (r0b, r1b, r2b, r3b)
    sg = (sg0, sg1, sg2, sg3)
    ss = (ss0, ss1, ss2, ss3)

    rb = s * _NB
    row0 = s * _RPT
    zero16 = jnp.zeros((16,), jnp.float32)

    def chunk_body(kk, cc):
        k = c * 2 + kk
        koff = (k * _NPAD).astype(jnp.int32)

        def zr(i, c2):
            for j8 in range(8):
                r0b[i, pl.ds(j8 * 16, 16)] = zero16
            return c2

        lax.fori_loop(0, _EB, zr, 0)
        for m in range(_RPT // _EB):
            pltpu.sync_copy(r0b, acc.at[pl.ds(row0 + m * _EB, _EB)])
        plsc.subcore_barrier()

        def g_start(j, b):
            pltpu.async_copy(ps.at[srck.at[j]], rows[b], sg[b])

        def g_wait(b):
            pltpu.make_async_copy(ps.at[pl.ds(0, _EB)], rows[b],
                                  sg[b]).wait()

        def s_start(j, b):
            pltpu.async_copy(rows[b], acc.at[dstt.at[j]], ss[b], add=True)

        def s_wait(b):
            pltpu.make_async_copy(ps.at[pl.ds(0, _EB)], rows[b],
                                  ss[b]).wait()

        for seg in range(_NSEG):
            sb = rb + seg * _SB
            pltpu.async_copy(src2.at[pl.ds(sb, _SB)], srck, sg[0])
            pltpu.async_copy(dst2.at[pl.ds(sb, _SB)], dstt, sg[1])
            pltpu.async_copy(ew2.at[pl.ds(sb, _SB)], ewt, sg[2])
            pltpu.make_async_copy(src2.at[pl.ds(sb, _SB)], srck,
                                  sg[0]).wait()
            pltpu.make_async_copy(dst2.at[pl.ds(sb, _SB)], dstt,
                                  sg[1]).wait()
            pltpu.make_async_copy(ew2.at[pl.ds(sb, _SB)], ewt,
                                  sg[2]).wait()

            kv = jnp.full((16,), 0, jnp.int32) + koff

            def addk(i, c2):
                for g in range(_EB // 16):
                    srck[i, pl.ds(g * 16, 16)] = (
                        srck[i, pl.ds(g * 16, 16)] + kv)
                return c2

            lax.fori_loop(0, _SB, addk, 0)

            g_start(0, 0)
            g_start(1, 1)

            def ring_iter(it, c2):
                for b in range(4):
                    j = it * 4 + b
                    nb = (b + 2) % 4

                    @pl.when((j >= 2) & (j + 2 < _SB))
                    def _sw():
                        s_wait(nb)

                    @pl.when(j + 2 < _SB)
                    def _gs():
                        g_start(j + 2, nb)

                    g_wait(b)

                    def grp(g, c3):
                        base = pl.multiple_of(g * 16, 16)
                        wvec = ewt[j, pl.ds(base, 16)]
                        buf = rows[b]
                        for i16 in range(16):
                            wv = jnp.broadcast_to(wvec[i16:i16 + 1], (16,))
                            r = base + i16
                            for j8 in range(8):
                                buf[r, pl.ds(j8 * 16, 16)] = (
                                    buf[r, pl.ds(j8 * 16, 16)] * wv)
                        return c3

                    lax.fori_loop(0, _EB // 16, grp, 0)
                    s_start(j, b)
                return c2

            lax.fori_loop(0, _SB // 4, ring_iter, 0)
            for b in range(4):
                s_wait(b)

        plsc.subcore_barrier()
        for m in range(_RPT // _EB):
            r0 = row0 + m * _EB
            pltpu.sync_copy(acc.at[pl.ds(r0, _EB)], r0b)
            pltpu.sync_copy(r0b, outs.at[pl.ds(koff + r0, _EB)])
        return cc

    lax.fori_loop(0, _NPLANE // 2, chunk_body, 0)


_seg = pl.kernel(
    _seg_body,
    out_type=jax.ShapeDtypeStruct((_NPLANE * _NPAD, _FC), jnp.float32),
    mesh=plsc.VectorSubcoreMesh(core_axis_name="c", subcore_axis_name="s"),
    scratch_types=[
        pltpu.VMEM((_SB, _EB), jnp.int32),
        pltpu.VMEM((_SB, _EB), jnp.int32),
        pltpu.VMEM((_SB, _EB), jnp.float32),
        pltpu.VMEM((_EB, _FC), jnp.float32),
        pltpu.VMEM((_EB, _FC), jnp.float32),
        pltpu.VMEM((_EB, _FC), jnp.float32),
        pltpu.VMEM((_EB, _FC), jnp.float32),
        pltpu.SemaphoreType.DMA,
        pltpu.SemaphoreType.DMA,
        pltpu.SemaphoreType.DMA,
        pltpu.SemaphoreType.DMA,
        pltpu.SemaphoreType.DMA,
        pltpu.SemaphoreType.DMA,
        pltpu.SemaphoreType.DMA,
        pltpu.SemaphoreType.DMA,
        pltpu.VMEM_SHARED((_NPAD, _FC), jnp.float32),
    ],
)


def _leaky(x):
    return jnp.where(x >= 0, x, 0.01 * x)


def _mm_in_body(x_ref, w_ref, o_ref):
    y = jnp.dot(x_ref[...], w_ref[...], preferred_element_type=jnp.float32)
    for kk in range(_NPLANE):
        o_ref[kk] = y[:, kk * _FC:(kk + 1) * _FC]


def _mm_mid_body(a_ref, b_ref, w_ref, o_ref):
    h = jnp.concatenate([a_ref[kk] for kk in range(_NPLANE)], axis=1)
    h = _leaky(h + b_ref[...])
    y = jnp.dot(h, w_ref[...], preferred_element_type=jnp.float32)
    for kk in range(_NPLANE):
        o_ref[kk] = y[:, kk * _FC:(kk + 1) * _FC]


def _mm_out_body(a_ref, b_ref, w_ref, bl_ref, o_ref):
    h = jnp.concatenate([a_ref[kk] for kk in range(_NPLANE)], axis=1)
    h = _leaky(h + b_ref[...])
    o_ref[...] = (jnp.dot(h, w_ref[...], preferred_element_type=jnp.float32)
                  + bl_ref[...])


_STACK_SPEC = pl.BlockSpec((_NPLANE, _BM, _FC), lambda i: (0, i, 0))


def _mm_in(x, w):
    kin = x.shape[1]
    return pl.pallas_call(
        _mm_in_body,
        grid=(_N // _BM,),
        in_specs=[
            pl.BlockSpec((_BM, kin), lambda i: (i, 0)),
            pl.BlockSpec((kin, w.shape[1]), lambda i: (0, 0)),
        ],
        out_specs=_STACK_SPEC,
        out_shape=jax.ShapeDtypeStruct((_NPLANE, _NPAD, _FC), jnp.float32),
    )(x, w)


def _mm_mid(a, b, w):
    return pl.pallas_call(
        _mm_mid_body,
        grid=(_N // _BM,),
        in_specs=[
            _STACK_SPEC,
            pl.BlockSpec((1, b.shape[1]), lambda i: (0, 0)),
            pl.BlockSpec((w.shape[0], w.shape[1]), lambda i: (0, 0)),
        ],
        out_specs=_STACK_SPEC,
        out_shape=jax.ShapeDtypeStruct((_NPLANE, _NPAD, _FC), jnp.float32),
    )(a, b, w)


def _mm_out(a, b, w, bl):
    ncls = w.shape[1]
    return pl.pallas_call(
        _mm_out_body,
        grid=(_N // _BM,),
        in_specs=[
            _STACK_SPEC,
            pl.BlockSpec((1, b.shape[1]), lambda i: (0, 0)),
            pl.BlockSpec((w.shape[0], ncls), lambda i: (0, 0)),
            pl.BlockSpec((1, ncls), lambda i: (0, 0)),
        ],
        out_specs=pl.BlockSpec((_BM, ncls), lambda i: (i, 0)),
        out_shape=jax.ShapeDtypeStruct((_N, ncls), jnp.float32),
    )(a, b, w, bl)


def kernel(x, edge_index, aw0, aw1, W1, b1, W2, b2, Wl, bl):
    npad = _EROWS * _EB - _E
    src2 = jnp.pad(edge_index[0], (0, npad)).reshape(_EROWS, _EB)
    dst2 = jnp.pad(edge_index[1], (0, npad)).reshape(_EROWS, _EB)
    ew0 = jnp.pad(aw0, (0, npad)).reshape(_EROWS, _EB)
    ew1 = jnp.pad(aw1, (0, npad)).reshape(_EROWS, _EB)
    flat = (_NPLANE * _NPAD, _FC)
    stck = (_NPLANE, _NPAD, _FC)
    p = _mm_in(x, W1)
    a = _seg(p.reshape(flat), src2, dst2, ew0)
    q = _mm_mid(a.reshape(stck), b1.reshape(1, -1), W2)
    g = _seg(q.reshape(flat), src2, dst2, ew1)
    return _mm_out(g.reshape(stck), b2.reshape(1, -1), Wl, bl.reshape(1, -1))
